# Initial kernel scaffold; baseline (speedup 1.0000x reference)
import jax
import jax.numpy as jnp
from jax.experimental import pallas as pl


def kernel(x, W):
    raise NotImplementedError("write your pallas kernel here")


# speedup vs baseline: 1.8765x; 1.8765x over previous
import functools

import jax
import jax.numpy as jnp
from jax import lax
from jax.experimental import pallas as pl
from jax.experimental.pallas import tpu as pltpu
from jax.experimental.pallas import tpu_sc as plsc

_D = 64
_B = 16384 * 50
_NC = 2
_NS = 16
_NW = _NC * _NS
_BPW = _B // _NW
_GS = 128
_R = 1024
_GPB = _R // _GS
_NBLK = _BPW // _R


@functools.partial(
    pl.kernel,
    out_type=jax.ShapeDtypeStruct((_B, _D), jnp.float32),
    mesh=plsc.VectorSubcoreMesh(core_axis_name="c", subcore_axis_name="s"),
    compiler_params=pltpu.CompilerParams(use_tc_tiling_on_sc=False),
    scratch_types=[
        pltpu.VMEM((_BPW,), jnp.int32),
        pltpu.VMEM((_R, _D), jnp.float32),
        pltpu.SemaphoreType.DMA,
    ],
)
def _emb_gather(x_hbm, w_hbm, out_hbm, idx_v, rows_v, gsem):
    wid = lax.axis_index("s") * _NC + lax.axis_index("c")
    base = wid * _BPW
    pltpu.sync_copy(x_hbm.at[pl.ds(base, _BPW)], idx_v)

    def block(i, carry):
        waits = []
        for g in range(_GPB):
            src = w_hbm.at[idx_v.at[pl.ds(i * _R + g * _GS, _GS)]]
            dst = rows_v.at[pl.ds(g * _GS, _GS)]
            waits.append(pltpu.async_copy(src, dst, gsem))
        for w in waits:
            w.wait()
        pltpu.sync_copy(rows_v, out_hbm.at[pl.ds(base + i * _R, _R)])
        return carry

    lax.fori_loop(0, _NBLK, block, 0)


def kernel(x, W):
    out = _emb_gather(x.reshape(-1), W)
    return out.reshape(*x.shape, _D)


# speedup vs baseline: 1.8808x; 1.0023x over previous
import functools

import jax
import jax.numpy as jnp
from jax import lax
from jax.experimental import pallas as pl
from jax.experimental.pallas import tpu as pltpu
from jax.experimental.pallas import tpu_sc as plsc

_D = 64
_B = 16384 * 50
_NC = 2
_NS = 16
_NW = _NC * _NS
_BPW = _B // _NW
_GS = 128
_R = 512
_GPB = _R // _GS
_NBLK = _BPW // _R


@functools.partial(
    pl.kernel,
    out_type=jax.ShapeDtypeStruct((_B, _D), jnp.float32),
    mesh=plsc.VectorSubcoreMesh(core_axis_name="c", subcore_axis_name="s"),
    compiler_params=pltpu.CompilerParams(use_tc_tiling_on_sc=False),
    scratch_types=[
        pltpu.VMEM((_BPW,), jnp.int32),
        pltpu.VMEM((2, _R, _D), jnp.float32),
        pltpu.SemaphoreType.DMA,
        pltpu.SemaphoreType.DMA,
    ],
)
def _emb_gather(x_hbm, w_hbm, out_hbm, idx_v, rows_v, gsem, ssem):
    wid = lax.axis_index("s") * _NC + lax.axis_index("c")
    base = wid * _BPW
    pltpu.sync_copy(x_hbm.at[pl.ds(base, _BPW)], idx_v)

    def fire_gathers(i, b):
        waits = []
        for g in range(_GPB):
            src = w_hbm.at[idx_v.at[pl.ds(i * _R + g * _GS, _GS)]]
            dst = rows_v.at[b, pl.ds(g * _GS, _GS)]
            waits.append(pltpu.async_copy(src, dst, gsem))
        return waits

    def fire_store(i, b):
        pltpu.async_copy(rows_v.at[b], out_hbm.at[pl.ds(base + i * _R, _R)], ssem)

    def wait_store():
        pltpu.make_async_copy(rows_v.at[0], out_hbm.at[pl.ds(base, _R)], ssem).wait()

    w0 = fire_gathers(0, 0)
    w1 = fire_gathers(1, 1)
    for w in w0:
        w.wait()
    fire_store(0, 0)
    for w in w1:
        w.wait()
    fire_store(1, 1)

    def body(io, carry):
        for b in range(2):
            i = io * 2 + b
            wait_store()
            ws = fire_gathers(i, b)
            for w in ws:
                w.wait()
            fire_store(i, b)
        return carry

    lax.fori_loop(1, _NBLK // 2, body, 0)

    for _ in range(2):
        wait_store()


def kernel(x, W):
    out = _emb_gather(x.reshape(-1), W)
    return out.reshape(*x.shape, _D)
